# baseline (speedup 1.0000x reference)
import functools

import jax
import jax.numpy as jnp
from jax import lax
from jax.experimental import pallas as pl
from jax.experimental.pallas import tpu as pltpu
from jax.experimental.pallas import tpu_sc as plsc

B = 16384
NC = 1
NS = 16
NW = NC * NS
CHUNK = B // NW
H = CHUNK // 2
L = 16

_mesh = plsc.VectorSubcoreMesh(
    core_axis_name="c", subcore_axis_name="s", num_cores=NC
)


@functools.partial(
    pl.kernel,
    mesh=_mesh,
    out_type=jax.ShapeDtypeStruct((2, B), jnp.float32),
    scratch_types=[
        pltpu.VMEM((L,), jnp.float32),
        pltpu.VMEM((CHUNK,), jnp.int32),
        pltpu.VMEM((2, CHUNK), jnp.float32),
        pltpu.SemaphoreType.DMA,
        pltpu.SemaphoreType.DMA,
        pltpu.SemaphoreType.DMA,
        pltpu.SemaphoreType.DMA,
    ],
)
def _policy_sc(logits_hbm, state_hbm, out_hbm, tbl_v, st_v, o_v,
               sem_t, sem_a, sem_b, sem_o):
    wid = lax.axis_index("s") * NC + lax.axis_index("c")
    base = wid * CHUNK
    tbl_dma = pltpu.async_copy(logits_hbm, tbl_v, sem_t)
    in_dmas = [
        pltpu.async_copy(
            state_hbm.at[pl.ds(base + h * H, H)],
            st_v.at[pl.ds(h * H, H)],
            sem,
        )
        for h, sem in ((0, sem_a), (1, sem_b))
    ]
    tbl_dma.wait()
    s = 1.0 / (1.0 + jnp.exp(-tbl_v[...]))

    out_dmas = []
    for h in (0, 1):
        in_dmas[h].wait()
        for i in range(H // L):
            sl = pl.ds(h * H + i * L, L)
            p1 = s.at[st_v[sl]].get(mode="promise_in_bounds")
            o_v[1, sl] = p1
            o_v[0, sl] = 1.0 - p1
        out_dmas.append(
            pltpu.async_copy(
                o_v.at[:, pl.ds(h * H, H)],
                out_hbm.at[:, pl.ds(base + h * H, H)],
                sem_o,
            )
        )
    for dma in out_dmas:
        dma.wait()


def kernel(state, logits):
    logits16 = jnp.pad(logits.astype(jnp.float32), (0, L - 2))
    return _policy_sc(logits16, state.astype(jnp.int32))


# speedup vs baseline: 1.0274x; 1.0274x over previous
import functools

import jax
import jax.numpy as jnp
from jax import lax
from jax.experimental import pallas as pl
from jax.experimental.pallas import tpu as pltpu
from jax.experimental.pallas import tpu_sc as plsc

B = 16384
NC = 1
NS = 16
NW = NC * NS
CHUNK = B // NW
H = CHUNK // 2
L = 16

_mesh = plsc.VectorSubcoreMesh(
    core_axis_name="c", subcore_axis_name="s", num_cores=NC
)


@functools.partial(
    pl.kernel,
    mesh=_mesh,
    out_type=jax.ShapeDtypeStruct((2, B), jnp.float32),
    scratch_types=[
        pltpu.VMEM((L,), jnp.float32),
        pltpu.VMEM((CHUNK,), jnp.int32),
        pltpu.VMEM((2, CHUNK), jnp.float32),
        pltpu.SemaphoreType.DMA,
        pltpu.SemaphoreType.DMA,
        pltpu.SemaphoreType.DMA,
        pltpu.SemaphoreType.DMA,
    ],
)
def _policy_sc(logits_hbm, state_hbm, out_hbm, tbl_v, st_v, o_v,
               sem_t, sem_a, sem_b, sem_o):
    wid = lax.axis_index("s") * NC + lax.axis_index("c")
    base = wid * CHUNK
    tbl_dma = pltpu.async_copy(logits_hbm, tbl_v, sem_t)
    in_dmas = [
        pltpu.async_copy(
            state_hbm.at[pl.ds(base + h * H, H)],
            st_v.at[pl.ds(h * H, H)],
            sem,
        )
        for h, sem in ((0, sem_a), (1, sem_b))
    ]
    tbl_dma.wait()
    s = 1.0 / (1.0 + jnp.exp(-tbl_v[...]))

    out_dmas = []
    for h in (0, 1):
        in_dmas[h].wait()

        @plsc.parallel_loop(0, H // L, unroll=4)
        def _body(i):
            sl = pl.ds(h * H + i * L, L)
            p1 = s.at[st_v[sl]].get(mode="promise_in_bounds")
            o_v[1, sl] = p1
            o_v[0, sl] = 1.0 - p1

        out_dmas.append(
            pltpu.async_copy(
                o_v.at[:, pl.ds(h * H, H)],
                out_hbm.at[:, pl.ds(base + h * H, H)],
                sem_o,
            )
        )
    for dma in out_dmas:
        dma.wait()


def kernel(state, logits):
    logits16 = jnp.pad(logits.astype(jnp.float32), (0, L - 2))
    return _policy_sc(logits16, state.astype(jnp.int32))
